# baseline (speedup 1.0000x reference)
import functools

import jax
import jax.numpy as jnp
from jax import lax
from jax.experimental import pallas as pl
from jax.experimental.pallas import tpu as pltpu
from jax.experimental.pallas import tpu_sc as plsc

N = 100000
D = 128
M = 50000
NC = 2
NS = 16
NW = NC * NS

UPD_W = 3128

ROWS_W = 3200
ROWS_LAST = N - (NW - 1) * ROWS_W
CHUNK = 200
NCH = ROWS_W // CHUNK
NCH_LAST = ROWS_LAST // CHUNK
NBUF = 3

NP = NW * ROWS_W
SEG = NP // NS

_mesh = plsc.VectorSubcoreMesh(core_axis_name="c", subcore_axis_name="s")


@functools.partial(
    pl.kernel,
    out_type=jax.ShapeDtypeStruct((N, D), jnp.float32),
    mesh=_mesh,
    scratch_types=[
        pltpu.VMEM((UPD_W,), jnp.int32),
        pltpu.VMEM((UPD_W,), jnp.int32),
        pltpu.VMEM((UPD_W,), jnp.int32),
        pltpu.VMEM((SEG,), jnp.int32),
        pltpu.VMEM_SHARED((NP,), jnp.int32),
        [pltpu.VMEM((CHUNK,), jnp.int32) for _ in range(NCH)],
        [pltpu.VMEM((CHUNK, D), jnp.float32) for _ in range(NBUF)],
        [pltpu.SemaphoreType.DMA for _ in range(NBUF)],
        [pltpu.SemaphoreType.DMA for _ in range(NBUF)],
        pltpu.SemaphoreType.DMA,
    ],
)
def _shuffle(idx_hbm, perm_hbm, x_hbm, out_hbm,
             upos_v, perm_v, usrc_v, iota_v, src_sh, idxs, bufs, gsem, wsem,
             sem):
    s_id = lax.axis_index("s")
    w = s_id * NC + lax.axis_index("c")

    seg_base = s_id * SEG

    @pl.loop(0, SEG // 16)
    def _(i):
        iota_v[pl.ds(i * 16, 16)] = lax.iota(jnp.int32, 16) + (
            seg_base + i * 16)

    pltpu.sync_copy(iota_v, src_sh.at[pl.ds(seg_base, SEG)])

    ubase = jnp.minimum(s_id * UPD_W, M - UPD_W)
    pltpu.sync_copy(idx_hbm.at[pl.ds(ubase, UPD_W)], upos_v)
    pltpu.sync_copy(perm_hbm.at[pl.ds(ubase, UPD_W)], perm_v)
    pltpu.async_copy(idx_hbm.at[perm_v], usrc_v, sem).wait()
    plsc.subcore_barrier()

    pltpu.sync_copy(usrc_v, src_sh.at[upos_v])
    plsc.subcore_barrier()

    base = w * ROWS_W
    nfull = jnp.where(w == NW - 1, NCH_LAST, NCH)

    for c in range(NCH):
        pltpu.async_copy(src_sh.at[pl.ds(base + c * CHUNK, CHUNK)],
                         idxs[c], sem)
    for c in range(NCH):
        pltpu.make_async_copy(src_sh.at[pl.ds(base, CHUNK)],
                              idxs[c], sem).wait()

    def retire(c):
        b = c % NBUF
        pltpu.make_async_copy(x_hbm.at[idxs[c]], bufs[b], gsem[b]).wait()
        pltpu.async_copy(bufs[b],
                         out_hbm.at[pl.ds(base + c * CHUNK, CHUNK)], wsem[b])

    for c in range(NCH):
        b = c % NBUF
        if c >= NBUF:
            @pl.when(c < nfull)
            def _(b=b):
                pltpu.make_async_copy(
                    bufs[b], out_hbm.at[pl.ds(base, CHUNK)], wsem[b]).wait()

        @pl.when(c < nfull)
        def _(b=b, c=c):
            pltpu.async_copy(x_hbm.at[idxs[c]], bufs[b], gsem[b])

        if c >= 1:
            @pl.when(c - 1 < nfull)
            def _(c=c):
                retire(c - 1)

    @pl.when(NCH - 1 < nfull)
    def _():
        retire(NCH - 1)

    for b in range(NBUF):
        pltpu.make_async_copy(
            bufs[b], out_hbm.at[pl.ds(base, CHUNK)], wsem[b]).wait()


def kernel(x, indices, shuffle_perm):
    idx = indices.astype(jnp.int32)
    perm = shuffle_perm.astype(jnp.int32)
    return _shuffle(idx, perm, x)


# speedup vs baseline: 1.0027x; 1.0027x over previous
import functools

import jax
import jax.numpy as jnp
from jax import lax
from jax.experimental import pallas as pl
from jax.experimental.pallas import tpu as pltpu
from jax.experimental.pallas import tpu_sc as plsc

N = 100000
D = 128
M = 50000
NC = 2
NS = 16
NW = NC * NS

UPD_W = 3128

ROWS_W = 3200
ROWS_LAST = N - (NW - 1) * ROWS_W
CHUNK = 200
NCH = ROWS_W // CHUNK
NCH_LAST = ROWS_LAST // CHUNK
NBUF = 4

NP = NW * ROWS_W
SEG = NP // NS

_mesh = plsc.VectorSubcoreMesh(core_axis_name="c", subcore_axis_name="s")


@functools.partial(
    pl.kernel,
    out_type=jax.ShapeDtypeStruct((N, D), jnp.float32),
    mesh=_mesh,
    scratch_types=[
        pltpu.VMEM((UPD_W,), jnp.int32),
        pltpu.VMEM((UPD_W,), jnp.int32),
        pltpu.VMEM((UPD_W,), jnp.int32),
        pltpu.VMEM((SEG,), jnp.int32),
        pltpu.VMEM_SHARED((NP,), jnp.int32),
        [pltpu.VMEM((CHUNK,), jnp.int32) for _ in range(NCH)],
        [pltpu.VMEM((CHUNK, D), jnp.float32) for _ in range(NBUF)],
        [pltpu.SemaphoreType.DMA for _ in range(NBUF)],
        [pltpu.SemaphoreType.DMA for _ in range(NBUF)],
        pltpu.SemaphoreType.DMA,
    ],
)
def _shuffle(idx_hbm, perm_hbm, x_hbm, out_hbm,
             upos_v, perm_v, usrc_v, iota_v, src_sh, idxs, bufs, gsem, wsem,
             sem):
    s_id = lax.axis_index("s")
    w = s_id * NC + lax.axis_index("c")

    seg_base = s_id * SEG

    @pl.loop(0, SEG // 16)
    def _(i):
        iota_v[pl.ds(i * 16, 16)] = lax.iota(jnp.int32, 16) + (
            seg_base + i * 16)

    pltpu.sync_copy(iota_v, src_sh.at[pl.ds(seg_base, SEG)])

    ubase = jnp.minimum(s_id * UPD_W, M - UPD_W)
    pltpu.sync_copy(idx_hbm.at[pl.ds(ubase, UPD_W)], upos_v)
    pltpu.sync_copy(perm_hbm.at[pl.ds(ubase, UPD_W)], perm_v)
    pltpu.async_copy(idx_hbm.at[perm_v], usrc_v, sem).wait()
    plsc.subcore_barrier()

    pltpu.sync_copy(usrc_v, src_sh.at[upos_v])
    plsc.subcore_barrier()

    base = w * ROWS_W
    nfull = jnp.where(w == NW - 1, NCH_LAST, NCH)

    for c in range(NCH):
        pltpu.async_copy(src_sh.at[pl.ds(base + c * CHUNK, CHUNK)],
                         idxs[c], sem)
    for c in range(NCH):
        pltpu.make_async_copy(src_sh.at[pl.ds(base, CHUNK)],
                              idxs[c], sem).wait()

    def retire(c):
        b = c % NBUF
        pltpu.make_async_copy(x_hbm.at[idxs[c]], bufs[b], gsem[b]).wait()
        pltpu.async_copy(bufs[b],
                         out_hbm.at[pl.ds(base + c * CHUNK, CHUNK)], wsem[b])

    for c in range(NCH):
        b = c % NBUF
        if c >= NBUF:
            @pl.when(c < nfull)
            def _(b=b):
                pltpu.make_async_copy(
                    bufs[b], out_hbm.at[pl.ds(base, CHUNK)], wsem[b]).wait()

        @pl.when(c < nfull)
        def _(b=b, c=c):
            pltpu.async_copy(x_hbm.at[idxs[c]], bufs[b], gsem[b])

        if c >= 1:
            @pl.when(c - 1 < nfull)
            def _(c=c):
                retire(c - 1)

    @pl.when(NCH - 1 < nfull)
    def _():
        retire(NCH - 1)

    for b in range(NBUF):
        pltpu.make_async_copy(
            bufs[b], out_hbm.at[pl.ds(base, CHUNK)], wsem[b]).wait()


def kernel(x, indices, shuffle_perm):
    idx = indices.astype(jnp.int32)
    perm = shuffle_perm.astype(jnp.int32)
    return _shuffle(idx, perm, x)


# speedup vs baseline: 1.0147x; 1.0120x over previous
import functools

import jax
import jax.numpy as jnp
from jax import lax
from jax.experimental import pallas as pl
from jax.experimental.pallas import tpu as pltpu
from jax.experimental.pallas import tpu_sc as plsc

N = 100000
D = 128
M = 50000
NC = 2
NS = 16
NW = NC * NS

NP = 100096
UPD_W = 3128
SEG = NP // NS

ROWS_W = 3128
ROWS_LAST = N - (NW - 1) * ROWS_W
CHUNK = 184
NCH = ROWS_W // CHUNK
TAIL = ROWS_LAST - (NCH - 1) * CHUNK
NBUF = 3
NSLOT = ((NCH + NBUF - 1) // NBUF) * NBUF

_mesh = plsc.VectorSubcoreMesh(core_axis_name="c", subcore_axis_name="s")


@functools.partial(
    pl.kernel,
    out_type=jax.ShapeDtypeStruct((N, D), jnp.float32),
    mesh=_mesh,
    scratch_types=[
        pltpu.VMEM((UPD_W,), jnp.int32),
        pltpu.VMEM((UPD_W,), jnp.int32),
        pltpu.VMEM((UPD_W,), jnp.int32),
        pltpu.VMEM((SEG,), jnp.int32),
        pltpu.VMEM_SHARED((NP,), jnp.int32),
        [pltpu.VMEM((CHUNK,), jnp.int32) for _ in range(NBUF)],
        [pltpu.VMEM((CHUNK, D), jnp.float32) for _ in range(NBUF)],
        [pltpu.SemaphoreType.DMA for _ in range(NBUF)],
        [pltpu.SemaphoreType.DMA for _ in range(NBUF)],
        pltpu.VMEM((TAIL,), jnp.int32),
        pltpu.VMEM((TAIL, D), jnp.float32),
        pltpu.SemaphoreType.DMA,
    ],
)
def _shuffle(idx_hbm, perm_hbm, x_hbm, out_hbm,
             upos_v, perm_v, usrc_v, iota_v, src_sh, idxs, bufs, gsem, wsem,
             idx_t, rows_t, sem):
    s_id = lax.axis_index("s")
    w = s_id * NC + lax.axis_index("c")

    seg_base = s_id * SEG

    @pl.loop(0, SEG // 16)
    def _(i):
        iota_v[pl.ds(i * 16, 16)] = lax.iota(jnp.int32, 16) + (
            seg_base + i * 16)

    pltpu.sync_copy(iota_v, src_sh.at[pl.ds(seg_base, SEG)])

    ubase = jnp.minimum(s_id * UPD_W, M - UPD_W)
    pltpu.sync_copy(idx_hbm.at[pl.ds(ubase, UPD_W)], upos_v)
    pltpu.sync_copy(perm_hbm.at[pl.ds(ubase, UPD_W)], perm_v)
    pltpu.async_copy(idx_hbm.at[perm_v], usrc_v, sem).wait()
    plsc.subcore_barrier()

    pltpu.sync_copy(usrc_v, src_sh.at[upos_v])
    plsc.subcore_barrier()

    base = w * ROWS_W
    nfull = jnp.where(w == NW - 1, NCH - 1, NCH)

    @pl.loop(0, NSLOT, step=NBUF)
    def _(c0):
        for b in range(NBUF):
            c = c0 + b

            @pl.when(jnp.logical_and(c >= NBUF, c < nfull))
            def _(b=b):
                pltpu.make_async_copy(
                    bufs[b], out_hbm.at[pl.ds(base, CHUNK)], wsem[b]).wait()

            @pl.when(c < nfull)
            def _(b=b, c=c):
                pltpu.sync_copy(src_sh.at[pl.ds(base + c * CHUNK, CHUNK)],
                                idxs[b])
                pltpu.async_copy(x_hbm.at[idxs[b]], bufs[b], gsem[b])

            b1 = (b - 1) % NBUF

            @pl.when(jnp.logical_and(c >= 1, c - 1 < nfull))
            def _(b1=b1, c=c):
                pltpu.make_async_copy(
                    x_hbm.at[idxs[b1]], bufs[b1], gsem[b1]).wait()
                pltpu.async_copy(
                    bufs[b1],
                    out_hbm.at[pl.ds(base + (c - 1) * CHUNK, CHUNK)],
                    wsem[b1])

    for b in range(NBUF):
        pltpu.make_async_copy(
            bufs[b], out_hbm.at[pl.ds(base, CHUNK)], wsem[b]).wait()

    @pl.when(w == NW - 1)
    def _():
        off = base + (NCH - 1) * CHUNK
        pltpu.sync_copy(src_sh.at[pl.ds(off, TAIL)], idx_t)
        pltpu.async_copy(x_hbm.at[idx_t], rows_t, sem).wait()
        pltpu.sync_copy(rows_t, out_hbm.at[pl.ds(off, TAIL)])


def kernel(x, indices, shuffle_perm):
    idx = indices.astype(jnp.int32)
    perm = shuffle_perm.astype(jnp.int32)
    return _shuffle(idx, perm, x)
